# Initial kernel scaffold; baseline (speedup 1.0000x reference)
import jax
import jax.numpy as jnp
from jax.experimental import pallas as pl


def kernel(x_h, adj, edge_index, pos_feat, W_init, b_init, Wg0, bg0, Wg1, bg1, Wg2, bg2, W_last, b_last):
    raise NotImplementedError("write your pallas kernel here")


# speedup vs baseline: 12.3692x; 12.3692x over previous
import functools

import jax
import jax.numpy as jnp
from jax import lax
from jax.experimental import pallas as pl
from jax.experimental.pallas import tpu as pltpu
from jax.experimental.pallas import tpu_sc as plsc

N = 10000
D = 128
E = 320000
C = 40
CHUNK = 128
NCHUNKS = E // CHUNK
NW = 32
FULLW, REMW = NCHUNKS // NW, NCHUNKS % NW
RB = 10
BR = N // RB

_HI = lax.Precision.HIGHEST


def _sc_worker_id():
    cid = lax.axis_index("c")
    sid = lax.axis_index("s")
    return cid, sid, sid * 2 + cid


def _sc_zero_acc(zeros_hbm, acc, sid):
    @pl.when(sid < 15)
    def _():
        pltpu.sync_copy(zeros_hbm.at[pl.ds(sid * 640, 640)],
                        acc.at[pl.ds(sid * 640, 640)])

    @pl.when(sid == 15)
    def _():
        pltpu.sync_copy(zeros_hbm.at[pl.ds(9600, 400)],
                        acc.at[pl.ds(9600, 400)])


def _sc_copy_out(acc, out_hbm, cid, sid):
    @pl.when(sid < 15)
    def _():
        pltpu.sync_copy(acc.at[pl.ds(sid * 640, 640)],
                        out_hbm.at[cid, pl.ds(sid * 640, 640)])

    @pl.when(sid == 15)
    def _():
        pltpu.sync_copy(acc.at[pl.ds(9600, 400)],
                        out_hbm.at[cid, pl.ds(9600, 400)])


def _sc_deg(dst2d, zeros128, ones128):
    mesh = plsc.VectorSubcoreMesh(core_axis_name="c", subcore_axis_name="s")

    def body(dst_hbm, zeros_hbm, ones_hbm, out_hbm, acc, didx, ones_v):
        cid, sid, gwid = _sc_worker_id()
        pltpu.sync_copy(ones_hbm, ones_v)
        _sc_zero_acc(zeros_hbm, acc, sid)
        plsc.subcore_barrier()

        nw = FULLW + jnp.where(gwid < REMW, 1, 0)

        def step(j, carry):
            c = gwid + NW * j
            pltpu.sync_copy(dst_hbm.at[c], didx)
            pltpu.sync_copy(ones_v, acc.at[didx], add=True)
            return carry

        lax.fori_loop(0, nw, step, 0)
        plsc.subcore_barrier()
        _sc_copy_out(acc, out_hbm, cid, sid)

    return pl.kernel(
        body,
        out_type=jax.ShapeDtypeStruct((2, N, D), jnp.float32),
        mesh=mesh,
        scratch_types=[
            pltpu.VMEM_SHARED((N, D), jnp.float32),
            pltpu.VMEM((CHUNK,), jnp.int32),
            pltpu.VMEM((CHUNK, D), jnp.float32),
        ],
    )(dst2d, zeros128, ones128)


def _sc_scatter(hp, src2d, dst2d, zeros128):
    mesh = plsc.VectorSubcoreMesh(core_axis_name="c", subcore_axis_name="s")

    def body(hp_hbm, src_hbm, dst_hbm, zeros_hbm, out_hbm,
             acc, sidx, didx, rows, sem):
        cid, sid, gwid = _sc_worker_id()
        _sc_zero_acc(zeros_hbm, acc, sid)
        plsc.subcore_barrier()

        nw = FULLW + jnp.where(gwid < REMW, 1, 0)

        def step(j, carry):
            c = gwid + NW * j
            pltpu.sync_copy(src_hbm.at[c], sidx)
            pltpu.sync_copy(dst_hbm.at[c], didx)
            pltpu.async_copy(hp_hbm.at[sidx], rows, sem).wait()
            pltpu.sync_copy(rows, acc.at[didx], add=True)
            return carry

        lax.fori_loop(0, nw, step, 0)
        plsc.subcore_barrier()
        _sc_copy_out(acc, out_hbm, cid, sid)

    return pl.kernel(
        body,
        out_type=jax.ShapeDtypeStruct((2, N, D), jnp.float32),
        mesh=mesh,
        scratch_types=[
            pltpu.VMEM_SHARED((N, D), jnp.float32),
            pltpu.VMEM((CHUNK,), jnp.int32),
            pltpu.VMEM((CHUNK,), jnp.int32),
            pltpu.VMEM((CHUNK, D), jnp.float32),
            pltpu.SemaphoreType.DMA,
        ],
    )(hp, src2d, dst2d, zeros128)


def _dis_from_deg(deg_ref):
    deg = 1.0 + deg_ref[0, :, 0:1] + deg_ref[1, :, 0:1]
    return lax.rsqrt(deg)


def _init_body(x_ref, w_ref, b_ref, o_ref):
    o_ref[...] = lax.dot_general(
        x_ref[...], w_ref[...], (((1,), (1,)), ((), ())),
        precision=_HI, preferred_element_type=jnp.float32) + b_ref[...]


def _scale_body(x_ref, w_ref, deg_ref, o_ref):
    h = lax.dot_general(
        x_ref[...], w_ref[...], (((1,), (1,)), ((), ())),
        precision=_HI, preferred_element_type=jnp.float32)
    o_ref[...] = h * _dis_from_deg(deg_ref)


def _combine_body(s_ref, hp_ref, deg_ref, b_ref, o_ref, *, relu):
    v = (s_ref[0] + s_ref[1] + hp_ref[...]) * _dis_from_deg(deg_ref) + b_ref[...]
    o_ref[...] = jnp.maximum(v, 0.0) if relu else v


def _final_body(x_ref, w_ref, b_ref, emb_ref, logp_ref):
    e = lax.dot_general(
        x_ref[...], w_ref[...], (((1,), (1,)), ((), ())),
        precision=_HI, preferred_element_type=jnp.float32) + b_ref[...]
    emb_ref[...] = e
    m = jnp.max(e, axis=1, keepdims=True)
    s = e - m
    logp_ref[...] = s - jnp.log(jnp.sum(jnp.exp(s), axis=1, keepdims=True))


_ROWS = pl.BlockSpec((BR, D), lambda i: (i, 0))
_WMAT = pl.BlockSpec((D, D), lambda i: (0, 0))
_BIAS = pl.BlockSpec((1, D), lambda i: (0, 0))
_DEG2 = pl.BlockSpec((2, BR, D), lambda i: (0, i, 0))
_S2 = pl.BlockSpec((2, BR, D), lambda i: (0, i, 0))


def _tc_init(x, w, b):
    return pl.pallas_call(
        _init_body, grid=(RB,),
        in_specs=[_ROWS, _WMAT, _BIAS], out_specs=_ROWS,
        out_shape=jax.ShapeDtypeStruct((N, D), jnp.float32),
    )(x, w, b)


def _tc_scale(x, w, deg2):
    return pl.pallas_call(
        _scale_body, grid=(RB,),
        in_specs=[_ROWS, _WMAT, _DEG2], out_specs=_ROWS,
        out_shape=jax.ShapeDtypeStruct((N, D), jnp.float32),
    )(x, w, deg2)


def _tc_combine(s2, hp, deg2, b, relu):
    return pl.pallas_call(
        functools.partial(_combine_body, relu=relu), grid=(RB,),
        in_specs=[_S2, _ROWS, _DEG2, _BIAS], out_specs=_ROWS,
        out_shape=jax.ShapeDtypeStruct((N, D), jnp.float32),
    )(s2, hp, deg2, b)


def _tc_final(x, w, b):
    outspec = pl.BlockSpec((BR, C), lambda i: (i, 0))
    return pl.pallas_call(
        _final_body, grid=(RB,),
        in_specs=[_ROWS, pl.BlockSpec((C, D), lambda i: (0, 0)),
                  pl.BlockSpec((1, C), lambda i: (0, 0))],
        out_specs=[outspec, outspec],
        out_shape=[jax.ShapeDtypeStruct((N, C), jnp.float32),
                   jax.ShapeDtypeStruct((N, C), jnp.float32)],
    )(x, w, b)


def kernel(x_h, adj, edge_index, pos_feat, W_init, b_init,
           Wg0, bg0, Wg1, bg1, Wg2, bg2, W_last, b_last):
    src2d = edge_index[0].reshape(NCHUNKS, CHUNK)
    dst2d = edge_index[1].reshape(NCHUNKS, CHUNK)
    zeros128 = jnp.zeros((N, D), jnp.float32)
    ones128 = jnp.ones((CHUNK, D), jnp.float32)

    deg2 = _sc_deg(dst2d, zeros128, ones128)
    x = _tc_init(x_h, W_init, b_init.reshape(1, D))
    for i, (W, b) in enumerate(((Wg0, bg0), (Wg1, bg1), (Wg2, bg2))):
        hp = _tc_scale(x, W, deg2)
        s2 = _sc_scatter(hp, src2d, dst2d, zeros128)
        x = _tc_combine(s2, hp, deg2, b.reshape(1, D), relu=(i < 2))
    emb, logp = _tc_final(x, W_last, b_last.reshape(1, C))
    return (emb, logp)
